# baseline (speedup 1.0000x reference)
import jax
import jax.numpy as jnp
from jax.experimental import pallas as pl
from jax.experimental.pallas import tpu as pltpu


def _fused_body(x_ref, w_ref, o_ref, mask_ref):
    K, D = w_ref.shape

    @pl.when(pl.program_id(0) == 0)
    def _():
        z = w_ref[...] * 0.5
        zmax = jnp.max(z, axis=-1, keepdims=True)
        tau0 = zmax - 1.0

        tau = tau0
        for _ in range(10):
            r = jnp.maximum(z - tau, 0.0)
            g = jnp.sum(r * r, axis=-1, keepdims=True) - 1.0
            dg = 2.0 * jnp.sum(r, axis=-1, keepdims=True)
            tau = tau + g / dg
        mask_ref[...] = jnp.maximum(z - tau, 0.0) ** 2

    xb = x_ref[...]
    for n in range(K):
        o_ref[:, n, :] = xb * mask_ref[n, :]


@jax.jit
def kernel(x, W):
    B, D = x.shape
    K, _ = W.shape
    BLK = 1024
    grid = (B // BLK,)
    return pl.pallas_call(
        _fused_body,
        grid=grid,
        in_specs=[
            pl.BlockSpec((BLK, D), lambda i: (i, 0)),
            pl.BlockSpec((K, D), lambda i: (0, 0)),
        ],
        out_specs=pl.BlockSpec((BLK, K, D), lambda i: (i, 0, 0)),
        out_shape=jax.ShapeDtypeStruct((B, K, D), x.dtype),
        scratch_shapes=[pltpu.VMEM((K, D), jnp.float32)],
    )(x, W)


# speedup vs baseline: 1.1008x; 1.1008x over previous
import jax
import jax.numpy as jnp
from jax.experimental import pallas as pl
from jax.experimental.pallas import tpu as pltpu

_BLK = 1024
_NCH = 4
_CH = _BLK // _NCH


def _fused_body(x_ref, w_ref, o_hbm, mask_ref, buf_ref, sems):
    K, D = w_ref.shape
    i = pl.program_id(0)
    nsteps = pl.num_programs(0)

    @pl.when(i == 0)
    def _():
        z = w_ref[...] * 0.5
        zmax = jnp.max(z, axis=-1, keepdims=True)
        tau = zmax - 1.0
        for _ in range(10):
            r = jnp.maximum(z - tau, 0.0)
            g = jnp.sum(r * r, axis=-1, keepdims=True) - 1.0
            dg = 2.0 * jnp.sum(r, axis=-1, keepdims=True)
            tau = tau + g / dg
        mask_ref[...] = jnp.maximum(z - tau, 0.0) ** 2

    for c in range(_NCH):
        row0 = i * _BLK + c * _CH
        copy = pltpu.make_async_copy(
            buf_ref.at[c], o_hbm.at[pl.ds(row0, _CH)], sems.at[c])

        @pl.when(i > 0)
        def _():
            copy.wait()

        xc = x_ref[pl.ds(c * _CH, _CH), :]
        for n in range(K):
            buf_ref[c, :, n, :] = xc * mask_ref[n, :]
        copy.start()

    @pl.when(i == nsteps - 1)
    def _():
        for c in range(_NCH):
            pltpu.make_async_copy(
                buf_ref.at[c],
                o_hbm.at[pl.ds(i * _BLK + c * _CH, _CH)],
                sems.at[c],
            ).wait()


@jax.jit
def kernel(x, W):
    B, D = x.shape
    K, _ = W.shape
    grid = (B // _BLK,)
    return pl.pallas_call(
        _fused_body,
        grid=grid,
        in_specs=[
            pl.BlockSpec((_BLK, D), lambda i: (i, 0)),
            pl.BlockSpec((K, D), lambda i: (0, 0)),
        ],
        out_specs=pl.BlockSpec(memory_space=pl.ANY),
        out_shape=jax.ShapeDtypeStruct((B, K, D), x.dtype),
        scratch_shapes=[
            pltpu.VMEM((K, D), jnp.float32),
            pltpu.VMEM((_NCH, _CH, K, D), jnp.float32),
            pltpu.SemaphoreType.DMA((_NCH,)),
        ],
    )(x, W)


# speedup vs baseline: 1.1089x; 1.0074x over previous
import jax
import jax.numpy as jnp
from jax.experimental import pallas as pl
from jax.experimental.pallas import tpu as pltpu

_BLK = 1024
_NCH = 8
_CH = _BLK // _NCH


def _fused_body(x_ref, w_ref, o_hbm, mask_ref, buf_ref, sems):
    K, D = w_ref.shape
    i = pl.program_id(0)
    nsteps = pl.num_programs(0)

    @pl.when(i == 0)
    def _():
        z = w_ref[...] * 0.5
        zmax = jnp.max(z, axis=-1, keepdims=True)
        tau = zmax - 1.0
        for _ in range(10):
            r = jnp.maximum(z - tau, 0.0)
            g = jnp.sum(r * r, axis=-1, keepdims=True) - 1.0
            dg = 2.0 * jnp.sum(r, axis=-1, keepdims=True)
            tau = tau + g / dg
        mask_ref[...] = jnp.maximum(z - tau, 0.0) ** 2

    for c in range(_NCH):
        row0 = i * _BLK + c * _CH
        copy = pltpu.make_async_copy(
            buf_ref.at[c], o_hbm.at[pl.ds(row0, _CH)], sems.at[c])

        @pl.when(i > 0)
        def _():
            copy.wait()

        xc = x_ref[pl.ds(c * _CH, _CH), :]
        for n in range(K):
            buf_ref[c, :, n, :] = xc * mask_ref[n, :]
        copy.start()

    @pl.when(i == nsteps - 1)
    def _():
        for c in range(_NCH):
            pltpu.make_async_copy(
                buf_ref.at[c],
                o_hbm.at[pl.ds(i * _BLK + c * _CH, _CH)],
                sems.at[c],
            ).wait()


@jax.jit
def kernel(x, W):
    B, D = x.shape
    K, _ = W.shape
    grid = (B // _BLK,)
    return pl.pallas_call(
        _fused_body,
        grid=grid,
        in_specs=[
            pl.BlockSpec((_BLK, D), lambda i: (i, 0)),
            pl.BlockSpec((K, D), lambda i: (0, 0)),
        ],
        out_specs=pl.BlockSpec(memory_space=pl.ANY),
        out_shape=jax.ShapeDtypeStruct((B, K, D), x.dtype),
        scratch_shapes=[
            pltpu.VMEM((K, D), jnp.float32),
            pltpu.VMEM((_NCH, _CH, K, D), jnp.float32),
            pltpu.SemaphoreType.DMA((_NCH,)),
        ],
    )(x, W)


# speedup vs baseline: 1.1107x; 1.0016x over previous
import jax
import jax.numpy as jnp
from jax.experimental import pallas as pl
from jax.experimental.pallas import tpu as pltpu

_BLK = 1024
_NCH = 16
_CH = _BLK // _NCH


def _fused_body(x_ref, w_ref, o_hbm, mask_ref, buf_ref, sems):
    K, D = w_ref.shape
    i = pl.program_id(0)
    nsteps = pl.num_programs(0)

    @pl.when(i == 0)
    def _():
        z = w_ref[...] * 0.5
        zmax = jnp.max(z, axis=-1, keepdims=True)
        tau = zmax - 1.0
        for _ in range(10):
            r = jnp.maximum(z - tau, 0.0)
            g = jnp.sum(r * r, axis=-1, keepdims=True) - 1.0
            dg = 2.0 * jnp.sum(r, axis=-1, keepdims=True)
            tau = tau + g / dg
        mask_ref[...] = jnp.maximum(z - tau, 0.0) ** 2

    for c in range(_NCH):
        row0 = i * _BLK + c * _CH
        copy = pltpu.make_async_copy(
            buf_ref.at[c], o_hbm.at[pl.ds(row0, _CH)], sems.at[c])

        @pl.when(i > 0)
        def _():
            copy.wait()

        xc = x_ref[pl.ds(c * _CH, _CH), :]
        for n in range(K):
            buf_ref[c, :, n, :] = xc * mask_ref[n, :]
        copy.start()

    @pl.when(i == nsteps - 1)
    def _():
        for c in range(_NCH):
            pltpu.make_async_copy(
                buf_ref.at[c],
                o_hbm.at[pl.ds(i * _BLK + c * _CH, _CH)],
                sems.at[c],
            ).wait()


@jax.jit
def kernel(x, W):
    B, D = x.shape
    K, _ = W.shape
    grid = (B // _BLK,)
    return pl.pallas_call(
        _fused_body,
        grid=grid,
        in_specs=[
            pl.BlockSpec((_BLK, D), lambda i: (i, 0)),
            pl.BlockSpec((K, D), lambda i: (0, 0)),
        ],
        out_specs=pl.BlockSpec(memory_space=pl.ANY),
        out_shape=jax.ShapeDtypeStruct((B, K, D), x.dtype),
        scratch_shapes=[
            pltpu.VMEM((K, D), jnp.float32),
            pltpu.VMEM((_NCH, _CH, K, D), jnp.float32),
            pltpu.SemaphoreType.DMA((_NCH,)),
        ],
    )(x, W)


# speedup vs baseline: 1.1245x; 1.0124x over previous
import jax
import jax.numpy as jnp
from jax.experimental import pallas as pl
from jax.experimental.pallas import tpu as pltpu

_BLK = 2048
_NCH = 16
_CH = _BLK // _NCH


def _fused_body(x_ref, w_ref, o_hbm, mask_ref, buf_ref, sems):
    K, D = w_ref.shape
    i = pl.program_id(0)
    nsteps = pl.num_programs(0)

    @pl.when(i == 0)
    def _():
        z = w_ref[...] * 0.5
        zmax = jnp.max(z, axis=-1, keepdims=True)
        tau = zmax - 1.0
        for _ in range(10):
            r = jnp.maximum(z - tau, 0.0)
            g = jnp.sum(r * r, axis=-1, keepdims=True) - 1.0
            dg = 2.0 * jnp.sum(r, axis=-1, keepdims=True)
            tau = tau + g / dg
        mask_ref[...] = jnp.maximum(z - tau, 0.0) ** 2

    for c in range(_NCH):
        row0 = i * _BLK + c * _CH
        copy = pltpu.make_async_copy(
            buf_ref.at[c], o_hbm.at[pl.ds(row0, _CH)], sems.at[c])

        @pl.when(i > 0)
        def _():
            copy.wait()

        xc = x_ref[pl.ds(c * _CH, _CH), :]
        for n in range(K):
            buf_ref[c, :, n, :] = xc * mask_ref[n, :]
        copy.start()

    @pl.when(i == nsteps - 1)
    def _():
        for c in range(_NCH):
            pltpu.make_async_copy(
                buf_ref.at[c],
                o_hbm.at[pl.ds(i * _BLK + c * _CH, _CH)],
                sems.at[c],
            ).wait()


@jax.jit
def kernel(x, W):
    B, D = x.shape
    K, _ = W.shape
    grid = (B // _BLK,)
    return pl.pallas_call(
        _fused_body,
        grid=grid,
        in_specs=[
            pl.BlockSpec((_BLK, D), lambda i: (i, 0)),
            pl.BlockSpec((K, D), lambda i: (0, 0)),
        ],
        out_specs=pl.BlockSpec(memory_space=pl.ANY),
        out_shape=jax.ShapeDtypeStruct((B, K, D), x.dtype),
        scratch_shapes=[
            pltpu.VMEM((K, D), jnp.float32),
            pltpu.VMEM((_NCH, _CH, K, D), jnp.float32),
            pltpu.SemaphoreType.DMA((_NCH,)),
        ],
    )(x, W)


# speedup vs baseline: 1.1253x; 1.0007x over previous
import jax
import jax.numpy as jnp
from jax.experimental import pallas as pl
from jax.experimental.pallas import tpu as pltpu

_BLK = 4096
_NCH = 32
_CH = _BLK // _NCH
_NSLOT = 16


def _fused_body(x_ref, w_ref, o_hbm, mask_ref, buf_ref, sems):
    K, D = w_ref.shape
    i = pl.program_id(0)
    nsteps = pl.num_programs(0)

    @pl.when(i == 0)
    def _():
        z = w_ref[...] * 0.5
        zmax = jnp.max(z, axis=-1, keepdims=True)
        tau = zmax - 1.0
        for _ in range(10):
            r = jnp.maximum(z - tau, 0.0)
            g = jnp.sum(r * r, axis=-1, keepdims=True) - 1.0
            dg = 2.0 * jnp.sum(r, axis=-1, keepdims=True)
            tau = tau + g / dg
        mask_ref[...] = jnp.maximum(z - tau, 0.0) ** 2

    for c in range(_NCH):
        s = c % _NSLOT
        row0 = i * _BLK + c * _CH
        copy = pltpu.make_async_copy(
            buf_ref.at[s], o_hbm.at[pl.ds(row0, _CH)], sems.at[s])

        if c >= _NSLOT:
            copy.wait()
        else:
            @pl.when(i > 0)
            def _():
                copy.wait()

        xc = x_ref[pl.ds(c * _CH, _CH), :]
        for n in range(K):
            buf_ref[s, :, n, :] = xc * mask_ref[n, :]
        copy.start()

    @pl.when(i == nsteps - 1)
    def _():
        for c in range(_NCH - _NSLOT, _NCH):
            pltpu.make_async_copy(
                buf_ref.at[c % _NSLOT],
                o_hbm.at[pl.ds(i * _BLK + c * _CH, _CH)],
                sems.at[c % _NSLOT],
            ).wait()


@jax.jit
def kernel(x, W):
    B, D = x.shape
    K, _ = W.shape
    grid = (B // _BLK,)
    return pl.pallas_call(
        _fused_body,
        grid=grid,
        in_specs=[
            pl.BlockSpec((_BLK, D), lambda i: (i, 0)),
            pl.BlockSpec((K, D), lambda i: (0, 0)),
        ],
        out_specs=pl.BlockSpec(memory_space=pl.ANY),
        out_shape=jax.ShapeDtypeStruct((B, K, D), x.dtype),
        scratch_shapes=[
            pltpu.VMEM((K, D), jnp.float32),
            pltpu.VMEM((_NSLOT, _CH, K, D), jnp.float32),
            pltpu.SemaphoreType.DMA((_NSLOT,)),
        ],
    )(x, W)


# speedup vs baseline: 1.1279x; 1.0023x over previous
import jax
import jax.numpy as jnp
from jax.experimental import pallas as pl
from jax.experimental.pallas import tpu as pltpu

_BLK = 2048
_NCH = 32
_CH = _BLK // _NCH
_NSLOT = 32


def _fused_body(x_ref, w_ref, o_hbm, mask_ref, buf_ref, sems):
    K, D = w_ref.shape
    i = pl.program_id(0)
    nsteps = pl.num_programs(0)

    @pl.when(i == 0)
    def _():
        z = w_ref[...] * 0.5
        zmax = jnp.max(z, axis=-1, keepdims=True)
        tau = zmax - 1.0
        for _ in range(10):
            r = jnp.maximum(z - tau, 0.0)
            g = jnp.sum(r * r, axis=-1, keepdims=True) - 1.0
            dg = 2.0 * jnp.sum(r, axis=-1, keepdims=True)
            tau = tau + g / dg
        mask_ref[...] = jnp.maximum(z - tau, 0.0) ** 2

    for c in range(_NCH):
        s = c % _NSLOT
        row0 = i * _BLK + c * _CH
        copy = pltpu.make_async_copy(
            buf_ref.at[s], o_hbm.at[pl.ds(row0, _CH)], sems.at[s])

        if c >= _NSLOT:
            copy.wait()
        else:
            @pl.when(i > 0)
            def _():
                copy.wait()

        xc = x_ref[pl.ds(c * _CH, _CH), :]
        for n in range(K):
            buf_ref[s, :, n, :] = xc * mask_ref[n, :]
        copy.start()

    @pl.when(i == nsteps - 1)
    def _():
        for c in range(_NCH - _NSLOT, _NCH):
            pltpu.make_async_copy(
                buf_ref.at[c % _NSLOT],
                o_hbm.at[pl.ds(i * _BLK + c * _CH, _CH)],
                sems.at[c % _NSLOT],
            ).wait()


@jax.jit
def kernel(x, W):
    B, D = x.shape
    K, _ = W.shape
    grid = (B // _BLK,)
    return pl.pallas_call(
        _fused_body,
        grid=grid,
        in_specs=[
            pl.BlockSpec((_BLK, D), lambda i: (i, 0)),
            pl.BlockSpec((K, D), lambda i: (0, 0)),
        ],
        out_specs=pl.BlockSpec(memory_space=pl.ANY),
        out_shape=jax.ShapeDtypeStruct((B, K, D), x.dtype),
        scratch_shapes=[
            pltpu.VMEM((K, D), jnp.float32),
            pltpu.VMEM((_NSLOT, _CH, K, D), jnp.float32),
            pltpu.SemaphoreType.DMA((_NSLOT,)),
        ],
    )(x, W)
